# Initial kernel scaffold; baseline (speedup 1.0000x reference)
import jax
import jax.numpy as jnp
from jax.experimental import pallas as pl


def kernel(features, segment_ids, w_gate, w_noise, expert_W, noise):
    raise NotImplementedError("write your pallas kernel here")


# speedup vs baseline: 6.1598x; 6.1598x over previous
import functools

import jax
import jax.numpy as jnp
from jax import lax
from jax.experimental import pallas as pl
from jax.experimental.pallas import tpu as pltpu
from jax.experimental.pallas import tpu_sc as plsc

_B = 1024
_N = 100000
_D = 128
_E = 8
_K = 2
_NOISE_EPS = 0.01

_NC = 2
_NS = 16
_NW = _NC * _NS
_PT = _N // _NW
_C = 125
_CP = 128
_NCH = _PT // _C
_PAD_ROWS = 8
_ACC_ROWS = _B + _PAD_ROWS


def _seg_sum_body(feat_hbm, ids_hbm, zeros_hbm, out_hbm,
                  ids_v, buf0, buf1, acc_sh, sem0, sem1):
    c = lax.axis_index("c")
    s = lax.axis_index("s")
    wid = c * _NS + s
    rows_per_tile = _B // _NS
    pltpu.sync_copy(zeros_hbm.at[pl.ds(s * rows_per_tile, rows_per_tile)],
                    acc_sh.at[pl.ds(s * rows_per_tile, rows_per_tile)])

    @pl.when(s == 0)
    def _():
        pltpu.sync_copy(zeros_hbm.at[pl.ds(0, _PAD_ROWS)],
                        acc_sh.at[pl.ds(_B, _PAD_ROWS)])

    pltpu.sync_copy(ids_hbm.at[wid], ids_v)
    plsc.subcore_barrier()

    base = wid * _PT
    bufs = (buf0, buf1)
    sems = (sem0, sem1)
    descs = [None, None]
    descs[0] = pltpu.async_copy(feat_hbm.at[pl.ds(base, _C)],
                                bufs[0].at[pl.ds(0, _C)], sems[0])
    for j in range(_NCH):
        cur = j % 2
        nxt = (j + 1) % 2
        if j + 1 < _NCH:
            descs[nxt] = pltpu.async_copy(
                feat_hbm.at[pl.ds(base + (j + 1) * _C, _C)],
                bufs[nxt].at[pl.ds(0, _C)], sems[nxt])
        descs[cur].wait()
        pltpu.sync_copy(bufs[cur], acc_sh.at[ids_v.at[j]], add=True)
    plsc.subcore_barrier()
    pltpu.sync_copy(acc_sh.at[pl.ds(s * rows_per_tile, rows_per_tile)],
                    out_hbm.at[c, pl.ds(s * rows_per_tile, rows_per_tile)])


@functools.lru_cache(maxsize=1)
def _make_seg_sum():
    return pl.kernel(
        _seg_sum_body,
        out_type=jax.ShapeDtypeStruct((_NC, _B, _D), jnp.float32),
        mesh=plsc.VectorSubcoreMesh(core_axis_name="c", subcore_axis_name="s",
                                    num_cores=_NC, num_subcores=_NS),
        compiler_params=pltpu.CompilerParams(use_tc_tiling_on_sc=False),
        scratch_types=[
            pltpu.VMEM((_NCH, _CP), jnp.int32),
            pltpu.VMEM((_CP, _D), jnp.float32),
            pltpu.VMEM((_CP, _D), jnp.float32),
            pltpu.VMEM_SHARED((_ACC_ROWS, _D), jnp.float32),
            pltpu.SemaphoreType.DMA,
            pltpu.SemaphoreType.DMA,
        ],
    )


def _cv_sq(x):
    m = jnp.sum(x) / _E
    v = jnp.sum((x - m) ** 2) / (_E - 1)
    return v / (m * m + 1e-10)


def _gate_expert_body(part_ref, wg_ref, wn_ref, ew_ref, noise_ref,
                      y_ref, gates_ref, loss_ref):
    feats = part_ref[0] + part_ref[1]
    clean = jnp.dot(feats, wg_ref[...], preferred_element_type=jnp.float32)
    raw = jnp.dot(feats, wn_ref[...], preferred_element_type=jnp.float32)
    mx = jnp.maximum(raw, 0.0)
    stddev = mx + jnp.log(jnp.exp(raw - mx) + jnp.exp(-mx)) + _NOISE_EPS
    noisy = clean + noise_ref[...] * stddev

    eidx = lax.broadcasted_iota(jnp.int32, (_B, _E), 1)
    neg = jnp.float32(-3e38)
    cur = noisy
    vals, idxs = [], []
    for _ in range(_K + 1):
        m = jnp.max(cur, axis=1, keepdims=True)
        i = jnp.min(jnp.where(cur == m, eidx, _E), axis=1, keepdims=True)
        vals.append(m)
        idxs.append(i)
        cur = jnp.where(eidx == i, neg, cur)
    v1, v2, v3 = vals
    i1, i2 = idxs[0], idxs[1]

    e2 = jnp.exp(v2 - v1)
    denom = 1.0 + e2
    gates = (jnp.where(eidx == i1, 1.0 / denom, 0.0)
             + jnp.where(eidx == i2, e2 / denom, 0.0))
    gates_ref[...] = gates

    inv_sqrt2 = jnp.float32(0.7071067811865476)
    is_in = noisy > v3
    p_in = 0.5 * (1.0 + lax.erf((clean - v3) * inv_sqrt2 / stddev))
    p_out = 0.5 * (1.0 + lax.erf((clean - v2) * inv_sqrt2 / stddev))
    load = jnp.sum(jnp.where(is_in, p_in, p_out), axis=0, keepdims=True)
    importance = jnp.sum(gates, axis=0, keepdims=True)
    loss_ref[...] = jnp.reshape(_cv_sq(importance) + _cv_sq(load), (1, 1))

    acc = jnp.zeros((_B, _D), jnp.float32)
    for e in range(_E):
        acc = acc + gates[:, e:e + 1] * jnp.dot(
            feats, ew_ref[e], preferred_element_type=jnp.float32)
    y_ref[...] = acc * (1.0 / _E)


def _gate_expert(partials, w_gate, w_noise, expert_W, noise):
    return pl.pallas_call(
        _gate_expert_body,
        out_shape=(
            jax.ShapeDtypeStruct((_B, _D), jnp.float32),
            jax.ShapeDtypeStruct((_B, _E), jnp.float32),
            jax.ShapeDtypeStruct((1, 1), jnp.float32),
        ),
    )(partials, w_gate, w_noise, expert_W, noise)


def kernel(features, segment_ids, w_gate, w_noise, expert_W, noise):
    ids = segment_ids.astype(jnp.int32).reshape(_NW, _NCH, _C)
    ids = jnp.pad(ids, ((0, 0), (0, 0), (0, _CP - _C)), constant_values=_B)
    zeros = jnp.zeros((_B, _D), jnp.float32)
    partials = _make_seg_sum()(features, ids, zeros)
    y, gates, loss = _gate_expert(partials, w_gate, w_noise, expert_W, noise)
    return y, gates, loss.reshape(())


# speedup vs baseline: 6.6003x; 1.0715x over previous
import functools

import jax
import jax.numpy as jnp
from jax import lax
from jax.experimental import pallas as pl
from jax.experimental.pallas import tpu as pltpu
from jax.experimental.pallas import tpu_sc as plsc

_B = 1024
_N = 100000
_D = 128
_E = 8
_K = 2
_NOISE_EPS = 0.01

_NC = 2
_NS = 16
_NW = _NC * _NS
_PT = _N // _NW
_C = 125
_CP = 128
_NCH = _PT // _C
_PAD_ROWS = 8
_ACC_ROWS = _B + _PAD_ROWS


_NBUF = 4
_PFD = 2


def _seg_sum_body(feat_hbm, ids_hbm, zeros_hbm, out_hbm,
                  ids_v, b0, b1, b2, b3, acc_sh,
                  gs0, gs1, gs2, gs3, ss0, ss1, ss2, ss3):
    c = lax.axis_index("c")
    s = lax.axis_index("s")
    wid = c * _NS + s
    base = wid * _PT
    bufs = (b0, b1, b2, b3)
    gsems = (gs0, gs1, gs2, gs3)
    ssems = (ss0, ss1, ss2, ss3)
    gd = [None] * _NBUF
    sd = [None] * _NBUF
    for b in range(_NBUF):
        gd[b] = pltpu.async_copy(feat_hbm.at[pl.ds(base + b * _C, _C)],
                                 bufs[b].at[pl.ds(0, _C)], gsems[b])

    rows_per_tile = _B // _NS
    pltpu.sync_copy(zeros_hbm.at[pl.ds(s * rows_per_tile, rows_per_tile)],
                    acc_sh.at[pl.ds(s * rows_per_tile, rows_per_tile)])

    @pl.when(s == 0)
    def _():
        pltpu.sync_copy(zeros_hbm.at[pl.ds(0, _PAD_ROWS)],
                        acc_sh.at[pl.ds(_B, _PAD_ROWS)])

    pltpu.sync_copy(ids_hbm.at[wid], ids_v)
    plsc.subcore_barrier()

    unwaited = set()
    for j in range(_NCH):
        sl = j % _NBUF
        gd[sl].wait()
        sd[sl] = pltpu.async_copy(bufs[sl], acc_sh.at[ids_v.at[j]],
                                  ssems[sl], add=True)
        unwaited.add(j)
        m = j + _PFD
        if _NBUF <= m < _NCH:
            psl = m % _NBUF
            sd[psl].wait()
            unwaited.discard(m - _NBUF)
            gd[psl] = pltpu.async_copy(
                feat_hbm.at[pl.ds(base + m * _C, _C)],
                bufs[psl].at[pl.ds(0, _C)], gsems[psl])
    for j in sorted(unwaited):
        sd[j % _NBUF].wait()
    plsc.subcore_barrier()
    pltpu.sync_copy(acc_sh.at[pl.ds(s * rows_per_tile, rows_per_tile)],
                    out_hbm.at[c, pl.ds(s * rows_per_tile, rows_per_tile)])


@functools.lru_cache(maxsize=1)
def _make_seg_sum():
    return pl.kernel(
        _seg_sum_body,
        out_type=jax.ShapeDtypeStruct((_NC, _B, _D), jnp.float32),
        mesh=plsc.VectorSubcoreMesh(core_axis_name="c", subcore_axis_name="s",
                                    num_cores=_NC, num_subcores=_NS),
        compiler_params=pltpu.CompilerParams(use_tc_tiling_on_sc=False),
        scratch_types=(
            [pltpu.VMEM((_NCH, _CP), jnp.int32)]
            + [pltpu.VMEM((_CP, _D), jnp.float32)] * _NBUF
            + [pltpu.VMEM_SHARED((_ACC_ROWS, _D), jnp.float32)]
            + [pltpu.SemaphoreType.DMA] * (2 * _NBUF)
        ),
    )


def _cv_sq(x):
    m = jnp.sum(x) / _E
    v = jnp.sum((x - m) ** 2) / (_E - 1)
    return v / (m * m + 1e-10)


def _gate_expert_body(part_ref, wg_ref, wn_ref, ew_ref, noise_ref,
                      y_ref, gates_ref, loss_ref):
    feats = part_ref[0] + part_ref[1]
    clean = jnp.dot(feats, wg_ref[...], preferred_element_type=jnp.float32)
    raw = jnp.dot(feats, wn_ref[...], preferred_element_type=jnp.float32)
    mx = jnp.maximum(raw, 0.0)
    stddev = mx + jnp.log(jnp.exp(raw - mx) + jnp.exp(-mx)) + _NOISE_EPS
    noisy = clean + noise_ref[...] * stddev

    eidx = lax.broadcasted_iota(jnp.int32, (_B, _E), 1)
    neg = jnp.float32(-3e38)
    cur = noisy
    vals, idxs = [], []
    for _ in range(_K + 1):
        m = jnp.max(cur, axis=1, keepdims=True)
        i = jnp.min(jnp.where(cur == m, eidx, _E), axis=1, keepdims=True)
        vals.append(m)
        idxs.append(i)
        cur = jnp.where(eidx == i, neg, cur)
    v1, v2, v3 = vals
    i1, i2 = idxs[0], idxs[1]

    e2 = jnp.exp(v2 - v1)
    denom = 1.0 + e2
    gates = (jnp.where(eidx == i1, 1.0 / denom, 0.0)
             + jnp.where(eidx == i2, e2 / denom, 0.0))
    gates_ref[...] = gates

    inv_sqrt2 = jnp.float32(0.7071067811865476)
    is_in = noisy > v3
    p_in = 0.5 * (1.0 + lax.erf((clean - v3) * inv_sqrt2 / stddev))
    p_out = 0.5 * (1.0 + lax.erf((clean - v2) * inv_sqrt2 / stddev))
    load = jnp.sum(jnp.where(is_in, p_in, p_out), axis=0, keepdims=True)
    importance = jnp.sum(gates, axis=0, keepdims=True)
    loss_ref[...] = jnp.reshape(_cv_sq(importance) + _cv_sq(load), (1, 1))

    acc = jnp.zeros((_B, _D), jnp.float32)
    for e in range(_E):
        acc = acc + gates[:, e:e + 1] * jnp.dot(
            feats, ew_ref[e], preferred_element_type=jnp.float32)
    y_ref[...] = acc * (1.0 / _E)


def _gate_expert(partials, w_gate, w_noise, expert_W, noise):
    return pl.pallas_call(
        _gate_expert_body,
        out_shape=(
            jax.ShapeDtypeStruct((_B, _D), jnp.float32),
            jax.ShapeDtypeStruct((_B, _E), jnp.float32),
            jax.ShapeDtypeStruct((1, 1), jnp.float32),
        ),
    )(partials, w_gate, w_noise, expert_W, noise)


def kernel(features, segment_ids, w_gate, w_noise, expert_W, noise):
    ids = segment_ids.astype(jnp.int32).reshape(_NW, _NCH, _C)
    ids = jnp.pad(ids, ((0, 0), (0, 0), (0, _CP - _C)), constant_values=_B)
    zeros = jnp.zeros((_B, _D), jnp.float32)
    partials = _make_seg_sum()(features, ids, zeros)
    y, gates, loss = _gate_expert(partials, w_gate, w_noise, expert_W, noise)
    return y, gates, loss.reshape(())
